# Initial kernel scaffold; baseline (speedup 1.0000x reference)
import jax
import jax.numpy as jnp
from jax.experimental import pallas as pl


def kernel(x, edge_index, type_lnc_idx, type_mi_idx, edge_label_index, W1c, b1c, W2c, b2c, Wl1, Wl2):
    raise NotImplementedError("write your pallas kernel here")


# speedup vs baseline: 4.2952x; 4.2952x over previous
import functools

import jax
import jax.numpy as jnp
from jax import lax
from jax.experimental import pallas as pl
from jax.experimental.pallas import tpu as pltpu
from jax.experimental.pallas import tpu_sc as plsc

N = 50000
E = 800000
EL = 100000
D = 100
DP = 128
CW = 8
NCH = 16
TRASH = N
SPROWS = 50176
ROWS_PT = 3136
PIECE = 392

ESTEPS = 391
EPAD = ESTEPS * 128

CSTEPS = 196
CPAD32 = 32 * CSTEPS * 128

ISTEPS = 5
IPAD = 16 * ISTEPS * 128

DSTEPS = 25
DPAD = 32 * DSTEPS * 128

_MESH = plsc.VectorSubcoreMesh(core_axis_name="c", subcore_axis_name="s",
                               num_cores=2, num_subcores=16)
_SC_PARAMS = pltpu.CompilerParams(use_tc_tiling_on_sc=False)
_f32 = jnp.float32


def _zero_vmem(ref, n):
    def body(i, _):
        ref[pl.ds(i * 16, 16)] = jnp.zeros((16,), _f32)
        return None
    lax.fori_loop(0, n // 16, body, None)


@functools.partial(
    pl.kernel,
    out_type=(
        jax.ShapeDtypeStruct((2 * SPROWS,), _f32),
        jax.ShapeDtypeStruct((SPROWS,), _f32),
        jax.ShapeDtypeStruct((SPROWS,), _f32),
    ),
    mesh=_MESH,
    compiler_params=_SC_PARAMS,
    scratch_types=(
        pltpu.VMEM((CSTEPS, 128), jnp.int32),
        pltpu.VMEM((ISTEPS, 128), jnp.int32),
        pltpu.VMEM((128,), _f32),
        pltpu.VMEM((ROWS_PT,), _f32),
        pltpu.VMEM_SHARED((SPROWS,), _f32),
        pltpu.VMEM_SHARED((SPROWS,), _f32),
    ),
)
def _sc_counts(dst_c, lnc_r, mi_r, degp, cnta, cntb,
               idx_v, sidx_v, ones_v, zbuf_v, sp_deg, sp_cnt):
    c = lax.axis_index("c")
    s = lax.axis_index("s")
    w = c * 16 + s

    def body16(i, _):
        ones_v[pl.ds(i * 16, 16)] = jnp.ones((16,), _f32)
        return None
    lax.fori_loop(0, 8, body16, None)
    _zero_vmem(zbuf_v, ROWS_PT)

    sl = pl.ds(s * ROWS_PT, ROWS_PT)
    pltpu.sync_copy(zbuf_v, sp_deg.at[sl])
    pltpu.sync_copy(zbuf_v, sp_cnt.at[sl])
    pltpu.sync_copy(dst_c.at[pl.ds(w * CSTEPS, CSTEPS)], idx_v)
    plsc.subcore_barrier()

    def deg_body(j, _):
        pltpu.sync_copy(ones_v, sp_deg.at[idx_v.at[j]], add=True)
        return None
    lax.fori_loop(0, CSTEPS, deg_body, None)

    @pl.when(c == 0)
    def _():
        pltpu.sync_copy(lnc_r.at[pl.ds(s * ISTEPS, ISTEPS)], sidx_v)

    @pl.when(c == 1)
    def _():
        pltpu.sync_copy(mi_r.at[pl.ds(s * ISTEPS, ISTEPS)], sidx_v)

    def cnt_body(j, _):
        pltpu.sync_copy(ones_v, sp_cnt.at[sidx_v.at[j]], add=True)
        return None
    lax.fori_loop(0, ISTEPS, cnt_body, None)

    plsc.subcore_barrier()
    pltpu.sync_copy(sp_deg.at[sl], zbuf_v)
    pltpu.sync_copy(zbuf_v, degp.at[pl.ds(c * SPROWS + s * ROWS_PT, ROWS_PT)])
    pltpu.sync_copy(sp_cnt.at[sl], zbuf_v)

    @pl.when(c == 0)
    def _():
        pltpu.sync_copy(zbuf_v, cnta.at[sl])

    @pl.when(c == 1)
    def _():
        pltpu.sync_copy(zbuf_v, cntb.at[sl])


@functools.partial(
    pl.kernel,
    out_type=tuple(jax.ShapeDtypeStruct((SPROWS, CW), _f32)
                   for _ in range(NCH)),
    mesh=_MESH,
    compiler_params=_SC_PARAMS,
    scratch_types=(
        pltpu.VMEM((ESTEPS, 128), jnp.int32),
        pltpu.VMEM((ESTEPS, 128), jnp.int32),
        pltpu.VMEM((128, CW), _f32),
        pltpu.VMEM((128, CW), _f32),
        pltpu.VMEM((PIECE, CW), _f32),
        pltpu.VMEM_SHARED((SPROWS, CW), _f32),
        pltpu.SemaphoreType.DMA,
        pltpu.SemaphoreType.DMA,
    ),
)
def _sc_edge(y0, y1, y2, y3, y4, y5, y6, y7,
             y8, y9, y10, y11, y12, y13, y14, y15, src_r, dst_r,
             a0, a1, a2, a3, a4, a5, a6, a7,
             a8, a9, a10, a11, a12, a13, a14, a15,
             src_v, dst_v, buf0, buf1, bounce, sp_acc, sem0, sem1):
    c = lax.axis_index("c")
    s = lax.axis_index("s")

    pltpu.sync_copy(src_r.at[pl.ds(s * ESTEPS, ESTEPS)], src_v)
    pltpu.sync_copy(dst_r.at[pl.ds(s * ESTEPS, ESTEPS)], dst_v)

    def run_chunk(y2d, out2d):
        for p in range(8):
            off = pl.ds(s * ROWS_PT + p * PIECE, PIECE)
            pltpu.sync_copy(y2d.at[off], bounce)
            pltpu.sync_copy(bounce, sp_acc.at[off])
        plsc.subcore_barrier()

        def gather(j, buf, sem):
            return pltpu.make_async_copy(y2d.at[src_v.at[j]], buf, sem)

        gather(0, buf0, sem0).start()

        def body(i, _):
            j0 = 2 * i
            j1 = 2 * i + 1
            gather(j1, buf1, sem1).start()
            gather(j0, buf0, sem0).wait()
            pltpu.sync_copy(buf0, sp_acc.at[dst_v.at[j0]], add=True)
            gather(j1 + 1, buf0, sem0).start()
            gather(j1, buf1, sem1).wait()
            pltpu.sync_copy(buf1, sp_acc.at[dst_v.at[j1]], add=True)
            return None
        lax.fori_loop(0, (ESTEPS - 1) // 2, body, None)
        jlast = ESTEPS - 1
        gather(jlast, buf0, sem0).wait()
        pltpu.sync_copy(buf0, sp_acc.at[dst_v.at[jlast]], add=True)

        plsc.subcore_barrier()
        for p in range(8):
            off = pl.ds(s * ROWS_PT + p * PIECE, PIECE)
            pltpu.sync_copy(sp_acc.at[off], bounce)
            pltpu.sync_copy(bounce, out2d.at[off])
        plsc.subcore_barrier()

    @pl.when(c == 0)
    def _():
        run_chunk(y0, a0)
        run_chunk(y1, a1)
        run_chunk(y2, a2)
        run_chunk(y3, a3)
        run_chunk(y4, a4)
        run_chunk(y5, a5)
        run_chunk(y6, a6)
        run_chunk(y7, a7)

    @pl.when(c == 1)
    def _():
        run_chunk(y8, a8)
        run_chunk(y9, a9)
        run_chunk(y10, a10)
        run_chunk(y11, a11)
        run_chunk(y12, a12)
        run_chunk(y13, a13)
        run_chunk(y14, a14)
        run_chunk(y15, a15)


@functools.partial(
    pl.kernel,
    out_type=jax.ShapeDtypeStruct((DPAD, DP), _f32),
    mesh=_MESH,
    compiler_params=_SC_PARAMS,
    scratch_types=(
        pltpu.VMEM((DSTEPS, 128), jnp.int32),
        pltpu.VMEM((DSTEPS, 128), jnp.int32),
        pltpu.VMEM((128, DP), _f32),
        pltpu.VMEM((128, DP), _f32),
        pltpu.SemaphoreType.DMA,
        pltpu.SemaphoreType.DMA,
    ),
)
def _sc_decode(qa, qb, el0_r, el1_r, x1p, i0_v, i1_v, bufa, bufb, sema, semb):
    c = lax.axis_index("c")
    s = lax.axis_index("s")
    w = c * 16 + s
    base = w * (DSTEPS * 128)

    pltpu.sync_copy(el0_r.at[pl.ds(w * DSTEPS, DSTEPS)], i0_v)
    pltpu.sync_copy(el1_r.at[pl.ds(w * DSTEPS, DSTEPS)], i1_v)

    def step(j, _):
        pltpu.async_copy(qa.at[i0_v.at[j]], bufa, sema)
        pltpu.async_copy(qb.at[i1_v.at[j]], bufb, semb)
        pltpu.make_async_copy(qa.at[i0_v.at[j]], bufa, sema).wait()
        pltpu.make_async_copy(qb.at[i1_v.at[j]], bufb, semb).wait()

        def row(r, _):
            for cc in range(DP // 16):
                sl = pl.ds(cc * 16, 16)
                bufa[r, sl] = jnp.maximum(bufa[r, sl] + bufb[r, sl], 0.0)
            return None
        lax.fori_loop(0, 128, row, None)
        pltpu.sync_copy(bufa, x1p.at[pl.ds(base + j * 128, 128)])
        return None
    lax.fori_loop(0, DSTEPS, step, None)


_RB = 400


def _tc_y1_body(x_ref, degt_ref, w_ref, y_ref, dinv_ref):
    deg = degt_ref[:, 0] + degt_ref[:, 1] + 1.0
    dinv = lax.rsqrt(deg)
    xw = jnp.dot(x_ref[...], w_ref[...], preferred_element_type=_f32)
    y_ref[...] = xw * dinv[:, None]
    dinv_ref[...] = dinv[:, None]


def _tc_y1(x, degt, w1p):
    return pl.pallas_call(
        _tc_y1_body,
        grid=(N // _RB,),
        in_specs=[
            pl.BlockSpec((_RB, D), lambda i: (i, 0)),
            pl.BlockSpec((_RB, 2), lambda i: (i, 0)),
            pl.BlockSpec((D, DP), lambda i: (0, 0)),
        ],
        out_specs=[
            pl.BlockSpec((_RB, DP), lambda i: (i, 0)),
            pl.BlockSpec((_RB, 1), lambda i: (i, 0)),
        ],
        out_shape=[
            jax.ShapeDtypeStruct((N, DP), _f32),
            jax.ShapeDtypeStruct((N, 1), _f32),
        ],
    )(x, degt, w1p)


def _tc_h_body(acc_ref, dinv_ref, b_ref, ca_ref, cb_ref,
               h_ref, sab_ref, cross_ref):
    hb = jnp.maximum(acc_ref[...] * dinv_ref[...] + b_ref[...], 0.0)
    h_ref[...] = hb
    ca = ca_ref[...]
    cb = cb_ref[...]
    sa = jnp.dot(ca.T, hb, preferred_element_type=_f32)
    sb = jnp.dot(cb.T, hb, preferred_element_type=_f32)
    sab = jnp.concatenate([sa, sb], axis=0)
    fa = (ca > 0.0).astype(_f32)
    crossblk = jnp.sum(fa * cb)
    col = lax.broadcasted_iota(jnp.int32, (1, DP), 1)
    crossmat = jnp.where(col == 0, crossblk, 0.0)

    @pl.when(pl.program_id(0) == 0)
    def _():
        sab_ref[...] = sab
        cross_ref[...] = crossmat

    @pl.when(pl.program_id(0) != 0)
    def _():
        sab_ref[...] += sab
        cross_ref[...] += crossmat


def _tc_h(accT, dinvc, b1p, ca2, cb2):
    return pl.pallas_call(
        _tc_h_body,
        grid=(N // _RB,),
        in_specs=[
            pl.BlockSpec((_RB, DP), lambda i: (i, 0)),
            pl.BlockSpec((_RB, 1), lambda i: (i, 0)),
            pl.BlockSpec((1, DP), lambda i: (0, 0)),
            pl.BlockSpec((_RB, 1), lambda i: (i, 0)),
            pl.BlockSpec((_RB, 1), lambda i: (i, 0)),
        ],
        out_specs=[
            pl.BlockSpec((_RB, DP), lambda i: (i, 0)),
            pl.BlockSpec((2, DP), lambda i: (0, 0)),
            pl.BlockSpec((1, DP), lambda i: (0, 0)),
        ],
        out_shape=[
            jax.ShapeDtypeStruct((N, DP), _f32),
            jax.ShapeDtypeStruct((2, DP), _f32),
            jax.ShapeDtypeStruct((1, DP), _f32),
        ],
    )(accT, dinvc, b1p, ca2, cb2)


def _tc_y2_body(h_ref, ca_ref, cb_ref, vab_ref, w_ref, dinv_ref, y2_ref):
    fa = (ca_ref[...] > 0.0).astype(_f32)
    fb = (cb_ref[...] > 0.0).astype(_f32)
    h2 = h_ref[...] + fa * vab_ref[0:1, :] + fb * vab_ref[1:2, :]
    y2_ref[...] = jnp.dot(h2, w_ref[...],
                          preferred_element_type=_f32) * dinv_ref[...]


def _tc_y2(h, ca2, cb2, vab, w2p, dinvc):
    return pl.pallas_call(
        _tc_y2_body,
        grid=(N // _RB,),
        in_specs=[
            pl.BlockSpec((_RB, DP), lambda i: (i, 0)),
            pl.BlockSpec((_RB, 1), lambda i: (i, 0)),
            pl.BlockSpec((_RB, 1), lambda i: (i, 0)),
            pl.BlockSpec((2, DP), lambda i: (0, 0)),
            pl.BlockSpec((DP, DP), lambda i: (0, 0)),
            pl.BlockSpec((_RB, 1), lambda i: (i, 0)),
        ],
        out_specs=pl.BlockSpec((_RB, DP), lambda i: (i, 0)),
        out_shape=jax.ShapeDtypeStruct((N, DP), _f32),
    )(h, ca2, cb2, vab, w2p, dinvc)


def _tc_qaqb_body(acc_ref, dinv_ref, b_ref, x_ref, wa_ref, wb_ref,
                  qa_ref, qb_ref):
    z = acc_ref[...] * dinv_ref[...] + b_ref[...]
    p = (z + x_ref[...]) * 0.5
    qa_ref[...] = jnp.dot(p, wa_ref[...], preferred_element_type=_f32)
    qb_ref[...] = jnp.dot(p, wb_ref[...], preferred_element_type=_f32)


def _tc_qaqb(acc2T, dinvc, b2p, xp, wap, wbp):
    return pl.pallas_call(
        _tc_qaqb_body,
        grid=(N // _RB,),
        in_specs=[
            pl.BlockSpec((_RB, DP), lambda i: (i, 0)),
            pl.BlockSpec((_RB, 1), lambda i: (i, 0)),
            pl.BlockSpec((1, DP), lambda i: (0, 0)),
            pl.BlockSpec((_RB, DP), lambda i: (i, 0)),
            pl.BlockSpec((DP, DP), lambda i: (0, 0)),
            pl.BlockSpec((DP, DP), lambda i: (0, 0)),
        ],
        out_specs=[
            pl.BlockSpec((_RB, DP), lambda i: (i, 0)),
            pl.BlockSpec((_RB, DP), lambda i: (i, 0)),
        ],
        out_shape=[
            jax.ShapeDtypeStruct((N, DP), _f32),
            jax.ShapeDtypeStruct((N, DP), _f32),
        ],
    )(acc2T, dinvc, b2p, xp, wap, wbp)


_RB2 = 1000


def _tc_score_body(x1_ref, w_ref, sc_ref, x1o_ref):
    x1 = x1_ref[...]
    sc_ref[...] = jnp.dot(x1, w_ref[...], preferred_element_type=_f32)
    x1o_ref[...] = x1[:, :D]


def _tc_score(x1p, wl2p):
    return pl.pallas_call(
        _tc_score_body,
        grid=(EL // _RB2,),
        in_specs=[
            pl.BlockSpec((_RB2, DP), lambda i: (i, 0)),
            pl.BlockSpec((DP, 8), lambda i: (0, 0)),
        ],
        out_specs=[
            pl.BlockSpec((_RB2, 8), lambda i: (i, 0)),
            pl.BlockSpec((_RB2, D), lambda i: (i, 0)),
        ],
        out_shape=[
            jax.ShapeDtypeStruct((EL, 8), _f32),
            jax.ShapeDtypeStruct((EL, D), _f32),
        ],
    )(x1p, wl2p)


def _pad_to(a, n, val):
    return jnp.concatenate(
        [a, jnp.full((n - a.shape[0],), val, a.dtype)])


def _chunked(y):
    yp = jnp.concatenate([y, jnp.zeros((SPROWS - N, DP), _f32)])
    y8 = jnp.transpose(yp.reshape(SPROWS, NCH, CW), (1, 0, 2))
    return tuple(y8[i] for i in range(NCH))


def _unchunk(accs):
    acc = jnp.stack(accs)
    return jnp.transpose(acc, (1, 0, 2)).reshape(SPROWS, DP)[:N]


def kernel(x, edge_index, type_lnc_idx, type_mi_idx, edge_label_index,
           W1c, b1c, W2c, b2c, Wl1, Wl2):
    src = edge_index[0]
    dst = edge_index[1]

    dst_c = _pad_to(dst, CPAD32, TRASH).reshape(32 * CSTEPS, 128)
    lnc_r = _pad_to(type_lnc_idx, IPAD, TRASH).reshape(16 * ISTEPS, 128)
    mi_r = _pad_to(type_mi_idx, IPAD, TRASH).reshape(16 * ISTEPS, 128)
    src_r = _pad_to(src, 16 * EPAD, 0).reshape(16 * ESTEPS, 128)
    dst_r = _pad_to(dst, 16 * EPAD, TRASH).reshape(16 * ESTEPS, 128)
    el0_r = _pad_to(edge_label_index[0], DPAD, 0).reshape(32 * DSTEPS, 128)
    el1_r = _pad_to(edge_label_index[1], DPAD, 0).reshape(32 * DSTEPS, 128)

    w1p = jnp.zeros((D, DP), _f32).at[:, :D].set(W1c)
    w2p = jnp.zeros((DP, DP), _f32).at[:D, :D].set(W2c)
    b1p = jnp.zeros((1, DP), _f32).at[0, :D].set(b1c)
    b2p = jnp.zeros((1, DP), _f32).at[0, :D].set(b2c)
    wap = jnp.zeros((DP, DP), _f32).at[:D, :D].set(Wl1[:D])
    wbp = jnp.zeros((DP, DP), _f32).at[:D, :D].set(Wl1[D:])
    wl2p = jnp.zeros((DP, 8), _f32).at[:D, 0:1].set(Wl2)
    xp = jnp.zeros((N, DP), _f32).at[:, :D].set(x)

    degp, cnta, cntb = _sc_counts(dst_c, lnc_r, mi_r)
    degt = jnp.transpose(degp.reshape(2, SPROWS)[:, :N], (1, 0))
    ca2 = cnta[:N, None]
    cb2 = cntb[:N, None]

    y1, dinvc = _tc_y1(x, degt, w1p)
    acc1 = _sc_edge(*_chunked(y1), src_r, dst_r)
    acc1T = _unchunk(acc1)
    h, sab, crossm = _tc_h(acc1T, dinvc, b1p, ca2, cb2)

    va = sab[0] / 10000.0
    cross = crossm[0, 0]
    vb = sab[1] / 10000.0 + va * (cross / 10000.0)
    vab = jnp.stack([va, vb])

    y2 = _tc_y2(h, ca2, cb2, vab, w2p, dinvc)
    acc2 = _sc_edge(*_chunked(y2), src_r, dst_r)
    acc2T = _unchunk(acc2)

    qa, qb = _tc_qaqb(acc2T, dinvc, b2p, xp, wap, wbp)
    x1pad = _sc_decode(qa, qb, el0_r, el1_r)
    x1full = x1pad[:EL]
    score8, x1 = _tc_score(x1full, wl2p)
    return score8[:, 0], x1


# speedup vs baseline: 5.1268x; 1.1936x over previous
import functools

import jax
import jax.numpy as jnp
from jax import lax
from jax.experimental import pallas as pl
from jax.experimental.pallas import tpu as pltpu
from jax.experimental.pallas import tpu_sc as plsc

N = 50000
E = 800000
EL = 100000
D = 100
DP = 128
CW = 8
NCH = 16
TRASH = N
SPROWS = 50176
ROWS_PT = 3136
PIECE = 392

ESTEPS = 391
EPAD = ESTEPS * 128

CSTEPS = 196
CPAD32 = 32 * CSTEPS * 128

ISTEPS = 5
IPAD = 16 * ISTEPS * 128

DSTEPS = 25
DPAD = 32 * DSTEPS * 128

_MESH = plsc.VectorSubcoreMesh(core_axis_name="c", subcore_axis_name="s",
                               num_cores=2, num_subcores=16)
_SC_PARAMS = pltpu.CompilerParams(use_tc_tiling_on_sc=False)
_f32 = jnp.float32


def _zero_vmem(ref, n):
    def body(i, _):
        ref[pl.ds(i * 16, 16)] = jnp.zeros((16,), _f32)
        return None
    lax.fori_loop(0, n // 16, body, None)


@functools.partial(
    pl.kernel,
    out_type=(
        jax.ShapeDtypeStruct((2 * SPROWS,), _f32),
        jax.ShapeDtypeStruct((SPROWS,), _f32),
        jax.ShapeDtypeStruct((SPROWS,), _f32),
    ),
    mesh=_MESH,
    compiler_params=_SC_PARAMS,
    scratch_types=(
        pltpu.VMEM((CSTEPS, 128), jnp.int32),
        pltpu.VMEM((ISTEPS, 128), jnp.int32),
        pltpu.VMEM((128,), _f32),
        pltpu.VMEM((ROWS_PT,), _f32),
        pltpu.VMEM_SHARED((SPROWS,), _f32),
        pltpu.VMEM_SHARED((SPROWS,), _f32),
    ),
)
def _sc_counts(dst_c, lnc_r, mi_r, degp, cnta, cntb,
               idx_v, sidx_v, ones_v, zbuf_v, sp_deg, sp_cnt):
    c = lax.axis_index("c")
    s = lax.axis_index("s")
    w = c * 16 + s

    def body16(i, _):
        ones_v[pl.ds(i * 16, 16)] = jnp.ones((16,), _f32)
        return None
    lax.fori_loop(0, 8, body16, None)
    _zero_vmem(zbuf_v, ROWS_PT)

    sl = pl.ds(s * ROWS_PT, ROWS_PT)
    pltpu.sync_copy(zbuf_v, sp_deg.at[sl])
    pltpu.sync_copy(zbuf_v, sp_cnt.at[sl])
    pltpu.sync_copy(dst_c.at[pl.ds(w * CSTEPS, CSTEPS)], idx_v)
    plsc.subcore_barrier()

    def deg_body(j, _):
        pltpu.sync_copy(ones_v, sp_deg.at[idx_v.at[j]], add=True)
        return None
    lax.fori_loop(0, CSTEPS, deg_body, None)

    @pl.when(c == 0)
    def _():
        pltpu.sync_copy(lnc_r.at[pl.ds(s * ISTEPS, ISTEPS)], sidx_v)

    @pl.when(c == 1)
    def _():
        pltpu.sync_copy(mi_r.at[pl.ds(s * ISTEPS, ISTEPS)], sidx_v)

    def cnt_body(j, _):
        pltpu.sync_copy(ones_v, sp_cnt.at[sidx_v.at[j]], add=True)
        return None
    lax.fori_loop(0, ISTEPS, cnt_body, None)

    plsc.subcore_barrier()
    pltpu.sync_copy(sp_deg.at[sl], zbuf_v)
    pltpu.sync_copy(zbuf_v, degp.at[pl.ds(c * SPROWS + s * ROWS_PT, ROWS_PT)])
    pltpu.sync_copy(sp_cnt.at[sl], zbuf_v)

    @pl.when(c == 0)
    def _():
        pltpu.sync_copy(zbuf_v, cnta.at[sl])

    @pl.when(c == 1)
    def _():
        pltpu.sync_copy(zbuf_v, cntb.at[sl])


@functools.partial(
    pl.kernel,
    out_type=jax.ShapeDtypeStruct((SPROWS, DP), _f32),
    mesh=_MESH,
    compiler_params=_SC_PARAMS,
    scratch_types=(
        pltpu.VMEM((ESTEPS, 128), jnp.int32),
        pltpu.VMEM((ESTEPS, 128), jnp.int32),
        pltpu.VMEM((128, CW), _f32),
        pltpu.VMEM((128, CW), _f32),
        pltpu.VMEM((PIECE, CW), _f32),
        pltpu.VMEM_SHARED((SPROWS, CW), _f32),
        pltpu.SemaphoreType.DMA,
        pltpu.SemaphoreType.DMA,
    ),
)
def _sc_edge(y0, y1, y2, y3, y4, y5, y6, y7,
             y8, y9, y10, y11, y12, y13, y14, y15, src_r, dst_r,
             out, src_v, dst_v, buf0, buf1, bounce, sp_acc, sem0, sem1):
    c = lax.axis_index("c")
    s = lax.axis_index("s")

    pltpu.sync_copy(src_r.at[pl.ds(s * ESTEPS, ESTEPS)], src_v)
    pltpu.sync_copy(dst_r.at[pl.ds(s * ESTEPS, ESTEPS)], dst_v)

    def run_chunk(y2d, q):
        for p in range(8):
            off = pl.ds(s * ROWS_PT + p * PIECE, PIECE)
            pltpu.sync_copy(y2d.at[off], bounce)
            pltpu.sync_copy(bounce, sp_acc.at[off])
        plsc.subcore_barrier()

        def gather(j, buf, sem):
            return pltpu.make_async_copy(y2d.at[src_v.at[j]], buf, sem)

        gather(0, buf0, sem0).start()

        def body(i, _):
            j0 = 2 * i
            j1 = 2 * i + 1
            gather(j1, buf1, sem1).start()
            gather(j0, buf0, sem0).wait()
            pltpu.sync_copy(buf0, sp_acc.at[dst_v.at[j0]], add=True)
            gather(j1 + 1, buf0, sem0).start()
            gather(j1, buf1, sem1).wait()
            pltpu.sync_copy(buf1, sp_acc.at[dst_v.at[j1]], add=True)
            return None
        lax.fori_loop(0, (ESTEPS - 1) // 2, body, None)
        jlast = ESTEPS - 1
        gather(jlast, buf0, sem0).wait()
        pltpu.sync_copy(buf0, sp_acc.at[dst_v.at[jlast]], add=True)

        plsc.subcore_barrier()
        csl = pl.ds(q * CW, CW)
        for p in range(8):
            off = pl.ds(s * ROWS_PT + p * PIECE, PIECE)
            pltpu.sync_copy(sp_acc.at[off], bounce)
            pltpu.sync_copy(bounce, out.at[off, csl])
        plsc.subcore_barrier()

    ys = (y0, y1, y2, y3, y4, y5, y6, y7,
          y8, y9, y10, y11, y12, y13, y14, y15)

    @pl.when(c == 0)
    def _():
        for q in range(8):
            run_chunk(ys[q], q)

    @pl.when(c == 1)
    def _():
        for q in range(8, 16):
            run_chunk(ys[q], q)


@functools.partial(
    pl.kernel,
    out_type=jax.ShapeDtypeStruct((DPAD, DP), _f32),
    mesh=_MESH,
    compiler_params=_SC_PARAMS,
    scratch_types=(
        pltpu.VMEM((DSTEPS, 128), jnp.int32),
        pltpu.VMEM((DSTEPS, 128), jnp.int32),
        pltpu.VMEM((128, DP), _f32),
        pltpu.VMEM((128, DP), _f32),
        pltpu.SemaphoreType.DMA,
        pltpu.SemaphoreType.DMA,
    ),
)
def _sc_decode(qa, qb, el0_r, el1_r, x1p, i0_v, i1_v, bufa, bufb, sema, semb):
    c = lax.axis_index("c")
    s = lax.axis_index("s")
    w = c * 16 + s
    base = w * (DSTEPS * 128)

    pltpu.sync_copy(el0_r.at[pl.ds(w * DSTEPS, DSTEPS)], i0_v)
    pltpu.sync_copy(el1_r.at[pl.ds(w * DSTEPS, DSTEPS)], i1_v)

    def step(j, _):
        pltpu.async_copy(qa.at[i0_v.at[j]], bufa, sema)
        pltpu.async_copy(qb.at[i1_v.at[j]], bufb, semb)
        pltpu.make_async_copy(qa.at[i0_v.at[j]], bufa, sema).wait()
        pltpu.make_async_copy(qb.at[i1_v.at[j]], bufb, semb).wait()

        def row(r, _):
            for cc in range(DP // 16):
                sl = pl.ds(cc * 16, 16)
                bufa[r, sl] = jnp.maximum(bufa[r, sl] + bufb[r, sl], 0.0)
            return None
        lax.fori_loop(0, 128, row, None)
        pltpu.sync_copy(bufa, x1p.at[pl.ds(base + j * 128, 128)])
        return None
    lax.fori_loop(0, DSTEPS, step, None)


_RB = 400


def _tc_y1_body(x_ref, degt_ref, w_ref, y_ref, dinv_ref):
    deg = degt_ref[:, 0] + degt_ref[:, 1] + 1.0
    dinv = lax.rsqrt(deg)
    xw = jnp.dot(x_ref[...], w_ref[...], preferred_element_type=_f32)
    y_ref[...] = xw * dinv[:, None]
    dinv_ref[...] = dinv[:, None]


def _tc_y1(x, degt, w1p):
    return pl.pallas_call(
        _tc_y1_body,
        grid=(N // _RB,),
        in_specs=[
            pl.BlockSpec((_RB, D), lambda i: (i, 0)),
            pl.BlockSpec((_RB, 2), lambda i: (i, 0)),
            pl.BlockSpec((D, DP), lambda i: (0, 0)),
        ],
        out_specs=[
            pl.BlockSpec((_RB, DP), lambda i: (i, 0)),
            pl.BlockSpec((_RB, 1), lambda i: (i, 0)),
        ],
        out_shape=[
            jax.ShapeDtypeStruct((N, DP), _f32),
            jax.ShapeDtypeStruct((N, 1), _f32),
        ],
    )(x, degt, w1p)


def _tc_h_body(acc_ref, dinv_ref, b_ref, ca_ref, cb_ref,
               h_ref, sab_ref, cross_ref):
    hb = jnp.maximum(acc_ref[...] * dinv_ref[...] + b_ref[...], 0.0)
    h_ref[...] = hb
    ca = ca_ref[...]
    cb = cb_ref[...]
    sa = jnp.dot(ca.T, hb, preferred_element_type=_f32)
    sb = jnp.dot(cb.T, hb, preferred_element_type=_f32)
    sab = jnp.concatenate([sa, sb], axis=0)
    fa = (ca > 0.0).astype(_f32)
    crossblk = jnp.sum(fa * cb)
    col = lax.broadcasted_iota(jnp.int32, (1, DP), 1)
    crossmat = jnp.where(col == 0, crossblk, 0.0)

    @pl.when(pl.program_id(0) == 0)
    def _():
        sab_ref[...] = sab
        cross_ref[...] = crossmat

    @pl.when(pl.program_id(0) != 0)
    def _():
        sab_ref[...] += sab
        cross_ref[...] += crossmat


def _tc_h(accT, dinvc, b1p, ca2, cb2):
    return pl.pallas_call(
        _tc_h_body,
        grid=(N // _RB,),
        in_specs=[
            pl.BlockSpec((_RB, DP), lambda i: (i, 0)),
            pl.BlockSpec((_RB, 1), lambda i: (i, 0)),
            pl.BlockSpec((1, DP), lambda i: (0, 0)),
            pl.BlockSpec((_RB, 1), lambda i: (i, 0)),
            pl.BlockSpec((_RB, 1), lambda i: (i, 0)),
        ],
        out_specs=[
            pl.BlockSpec((_RB, DP), lambda i: (i, 0)),
            pl.BlockSpec((2, DP), lambda i: (0, 0)),
            pl.BlockSpec((1, DP), lambda i: (0, 0)),
        ],
        out_shape=[
            jax.ShapeDtypeStruct((N, DP), _f32),
            jax.ShapeDtypeStruct((2, DP), _f32),
            jax.ShapeDtypeStruct((1, DP), _f32),
        ],
    )(accT, dinvc, b1p, ca2, cb2)


def _tc_y2_body(h_ref, ca_ref, cb_ref, vab_ref, w_ref, dinv_ref, y2_ref):
    fa = (ca_ref[...] > 0.0).astype(_f32)
    fb = (cb_ref[...] > 0.0).astype(_f32)
    h2 = h_ref[...] + fa * vab_ref[0:1, :] + fb * vab_ref[1:2, :]
    y2_ref[...] = jnp.dot(h2, w_ref[...],
                          preferred_element_type=_f32) * dinv_ref[...]


def _tc_y2(h, ca2, cb2, vab, w2p, dinvc):
    return pl.pallas_call(
        _tc_y2_body,
        grid=(N // _RB,),
        in_specs=[
            pl.BlockSpec((_RB, DP), lambda i: (i, 0)),
            pl.BlockSpec((_RB, 1), lambda i: (i, 0)),
            pl.BlockSpec((_RB, 1), lambda i: (i, 0)),
            pl.BlockSpec((2, DP), lambda i: (0, 0)),
            pl.BlockSpec((DP, DP), lambda i: (0, 0)),
            pl.BlockSpec((_RB, 1), lambda i: (i, 0)),
        ],
        out_specs=pl.BlockSpec((_RB, DP), lambda i: (i, 0)),
        out_shape=jax.ShapeDtypeStruct((N, DP), _f32),
    )(h, ca2, cb2, vab, w2p, dinvc)


def _tc_qaqb_body(acc_ref, dinv_ref, b_ref, x_ref, wa_ref, wb_ref,
                  qa_ref, qb_ref):
    z = acc_ref[...] * dinv_ref[...] + b_ref[...]
    p = (z + x_ref[...]) * 0.5
    qa_ref[...] = jnp.dot(p, wa_ref[...], preferred_element_type=_f32)
    qb_ref[...] = jnp.dot(p, wb_ref[...], preferred_element_type=_f32)


def _tc_qaqb(acc2T, dinvc, b2p, xp, wap, wbp):
    return pl.pallas_call(
        _tc_qaqb_body,
        grid=(N // _RB,),
        in_specs=[
            pl.BlockSpec((_RB, DP), lambda i: (i, 0)),
            pl.BlockSpec((_RB, 1), lambda i: (i, 0)),
            pl.BlockSpec((1, DP), lambda i: (0, 0)),
            pl.BlockSpec((_RB, DP), lambda i: (i, 0)),
            pl.BlockSpec((DP, DP), lambda i: (0, 0)),
            pl.BlockSpec((DP, DP), lambda i: (0, 0)),
        ],
        out_specs=[
            pl.BlockSpec((_RB, DP), lambda i: (i, 0)),
            pl.BlockSpec((_RB, DP), lambda i: (i, 0)),
        ],
        out_shape=[
            jax.ShapeDtypeStruct((N, DP), _f32),
            jax.ShapeDtypeStruct((N, DP), _f32),
        ],
    )(acc2T, dinvc, b2p, xp, wap, wbp)


_RB2 = 1000


def _tc_score_body(x1_ref, w_ref, sc_ref, x1o_ref):
    x1 = x1_ref[...]
    sc_ref[...] = jnp.dot(x1, w_ref[...], preferred_element_type=_f32)
    x1o_ref[...] = x1[:, :D]


def _tc_score(x1p, wl2p):
    return pl.pallas_call(
        _tc_score_body,
        grid=(EL // _RB2,),
        in_specs=[
            pl.BlockSpec((_RB2, DP), lambda i: (i, 0)),
            pl.BlockSpec((DP, 8), lambda i: (0, 0)),
        ],
        out_specs=[
            pl.BlockSpec((_RB2, 8), lambda i: (i, 0)),
            pl.BlockSpec((_RB2, D), lambda i: (i, 0)),
        ],
        out_shape=[
            jax.ShapeDtypeStruct((EL, 8), _f32),
            jax.ShapeDtypeStruct((EL, D), _f32),
        ],
    )(x1p, wl2p)


def _pad_to(a, n, val):
    return jnp.concatenate(
        [a, jnp.full((n - a.shape[0],), val, a.dtype)])


def _chunked(y):
    yp = jnp.concatenate([y, jnp.zeros((SPROWS - N, DP), _f32)])
    y8 = jnp.transpose(yp.reshape(SPROWS, NCH, CW), (1, 0, 2))
    return tuple(y8[i] for i in range(NCH))


def kernel(x, edge_index, type_lnc_idx, type_mi_idx, edge_label_index,
           W1c, b1c, W2c, b2c, Wl1, Wl2):
    src = edge_index[0]
    dst = edge_index[1]

    dst_c = _pad_to(dst, CPAD32, TRASH).reshape(32 * CSTEPS, 128)
    lnc_r = _pad_to(type_lnc_idx, IPAD, TRASH).reshape(16 * ISTEPS, 128)
    mi_r = _pad_to(type_mi_idx, IPAD, TRASH).reshape(16 * ISTEPS, 128)
    src_r = _pad_to(src, 16 * EPAD, 0).reshape(16 * ESTEPS, 128)
    dst_r = _pad_to(dst, 16 * EPAD, TRASH).reshape(16 * ESTEPS, 128)
    el0_r = _pad_to(edge_label_index[0], DPAD, 0).reshape(32 * DSTEPS, 128)
    el1_r = _pad_to(edge_label_index[1], DPAD, 0).reshape(32 * DSTEPS, 128)

    w1p = jnp.zeros((D, DP), _f32).at[:, :D].set(W1c)
    w2p = jnp.zeros((DP, DP), _f32).at[:D, :D].set(W2c)
    b1p = jnp.zeros((1, DP), _f32).at[0, :D].set(b1c)
    b2p = jnp.zeros((1, DP), _f32).at[0, :D].set(b2c)
    wap = jnp.zeros((DP, DP), _f32).at[:D, :D].set(Wl1[:D])
    wbp = jnp.zeros((DP, DP), _f32).at[:D, :D].set(Wl1[D:])
    wl2p = jnp.zeros((DP, 8), _f32).at[:D, 0:1].set(Wl2)
    xp = jnp.zeros((N, DP), _f32).at[:, :D].set(x)

    degp, cnta, cntb = _sc_counts(dst_c, lnc_r, mi_r)
    degt = jnp.transpose(degp.reshape(2, SPROWS)[:, :N], (1, 0))
    ca2 = cnta[:N, None]
    cb2 = cntb[:N, None]

    y1, dinvc = _tc_y1(x, degt, w1p)
    acc1T = _sc_edge(*_chunked(y1), src_r, dst_r)[:N]
    h, sab, crossm = _tc_h(acc1T, dinvc, b1p, ca2, cb2)

    va = sab[0] / 10000.0
    cross = crossm[0, 0]
    vb = sab[1] / 10000.0 + va * (cross / 10000.0)
    vab = jnp.stack([va, vb])

    y2 = _tc_y2(h, ca2, cb2, vab, w2p, dinvc)
    acc2T = _sc_edge(*_chunked(y2), src_r, dst_r)[:N]

    qa, qb = _tc_qaqb(acc2T, dinvc, b2p, xp, wap, wbp)
    x1pad = _sc_decode(qa, qb, el0_r, el1_r)
    x1full = x1pad[:EL]
    score8, x1 = _tc_score(x1full, wl2p)
    return score8[:, 0], x1


# speedup vs baseline: 5.6217x; 1.0965x over previous
import functools

import jax
import jax.numpy as jnp
from jax import lax
from jax.experimental import pallas as pl
from jax.experimental.pallas import tpu as pltpu
from jax.experimental.pallas import tpu_sc as plsc

N = 50000
E = 800000
EL = 100000
D = 100
DP = 128
CW = 8
NCH = 16
TRASH = N
SPROWS = 50176
ROWS_PT = 3136
PIECE = 392

ESTEPS = 391
EPAD = ESTEPS * 128

CSTEPS = 196
CPAD32 = 32 * CSTEPS * 128

ISTEPS = 5
IPAD = 16 * ISTEPS * 128

DSTEPS = 25
DPAD = 32 * DSTEPS * 128

_MESH = plsc.VectorSubcoreMesh(core_axis_name="c", subcore_axis_name="s",
                               num_cores=2, num_subcores=16)
_SC_PARAMS = pltpu.CompilerParams(use_tc_tiling_on_sc=False)
_f32 = jnp.float32


def _zero_vmem(ref, n):
    def body(i, _):
        ref[pl.ds(i * 16, 16)] = jnp.zeros((16,), _f32)
        return None
    lax.fori_loop(0, n // 16, body, None)


@functools.partial(
    pl.kernel,
    out_type=(
        jax.ShapeDtypeStruct((2 * SPROWS,), _f32),
        jax.ShapeDtypeStruct((SPROWS,), _f32),
        jax.ShapeDtypeStruct((SPROWS,), _f32),
    ),
    mesh=_MESH,
    compiler_params=_SC_PARAMS,
    scratch_types=(
        pltpu.VMEM((CSTEPS, 128), jnp.int32),
        pltpu.VMEM((ISTEPS, 128), jnp.int32),
        pltpu.VMEM((128,), _f32),
        pltpu.VMEM((ROWS_PT,), _f32),
        pltpu.VMEM_SHARED((SPROWS,), _f32),
        pltpu.VMEM_SHARED((SPROWS,), _f32),
    ),
)
def _sc_counts(dst_c, lnc_r, mi_r, degp, cnta, cntb,
               idx_v, sidx_v, ones_v, zbuf_v, sp_deg, sp_cnt):
    c = lax.axis_index("c")
    s = lax.axis_index("s")
    w = c * 16 + s

    def body16(i, _):
        ones_v[pl.ds(i * 16, 16)] = jnp.ones((16,), _f32)
        return None
    lax.fori_loop(0, 8, body16, None)
    _zero_vmem(zbuf_v, ROWS_PT)

    sl = pl.ds(s * ROWS_PT, ROWS_PT)
    pltpu.sync_copy(zbuf_v, sp_deg.at[sl])
    pltpu.sync_copy(zbuf_v, sp_cnt.at[sl])
    pltpu.sync_copy(dst_c.at[pl.ds(w * CSTEPS, CSTEPS)], idx_v)
    plsc.subcore_barrier()

    def deg_body(j, _):
        pltpu.sync_copy(ones_v, sp_deg.at[idx_v.at[j]], add=True)
        return None
    lax.fori_loop(0, CSTEPS, deg_body, None)

    @pl.when(c == 0)
    def _():
        pltpu.sync_copy(lnc_r.at[pl.ds(s * ISTEPS, ISTEPS)], sidx_v)

    @pl.when(c == 1)
    def _():
        pltpu.sync_copy(mi_r.at[pl.ds(s * ISTEPS, ISTEPS)], sidx_v)

    def cnt_body(j, _):
        pltpu.sync_copy(ones_v, sp_cnt.at[sidx_v.at[j]], add=True)
        return None
    lax.fori_loop(0, ISTEPS, cnt_body, None)

    plsc.subcore_barrier()
    pltpu.sync_copy(sp_deg.at[sl], zbuf_v)
    pltpu.sync_copy(zbuf_v, degp.at[pl.ds(c * SPROWS + s * ROWS_PT, ROWS_PT)])
    pltpu.sync_copy(sp_cnt.at[sl], zbuf_v)

    @pl.when(c == 0)
    def _():
        pltpu.sync_copy(zbuf_v, cnta.at[sl])

    @pl.when(c == 1)
    def _():
        pltpu.sync_copy(zbuf_v, cntb.at[sl])


@functools.partial(
    pl.kernel,
    out_type=jax.ShapeDtypeStruct((SPROWS, DP), _f32),
    mesh=_MESH,
    compiler_params=_SC_PARAMS,
    scratch_types=(
        pltpu.VMEM((ESTEPS, 128), jnp.int32),
        pltpu.VMEM((ESTEPS, 128), jnp.int32),
        pltpu.VMEM((128, CW), _f32),
        pltpu.VMEM((128, CW), _f32),
        pltpu.VMEM((PIECE, CW), _f32),
        pltpu.VMEM_SHARED((SPROWS, CW), _f32),
        pltpu.SemaphoreType.DMA,
        pltpu.SemaphoreType.DMA,
    ),
)
def _sc_edge(y0, y1, y2, y3, y4, y5, y6, y7,
             y8, y9, y10, y11, y12, y13, y14, y15, src_r, dst_r,
             out, src_v, dst_v, buf0, buf1, bounce, sp_acc, sem0, sem1):
    c = lax.axis_index("c")
    s = lax.axis_index("s")

    pltpu.sync_copy(src_r.at[pl.ds(s * ESTEPS, ESTEPS)], src_v)
    pltpu.sync_copy(dst_r.at[pl.ds(s * ESTEPS, ESTEPS)], dst_v)

    def run_chunk(y2d, q):
        for p in range(8):
            off = pl.ds(s * ROWS_PT + p * PIECE, PIECE)
            pltpu.sync_copy(y2d.at[off], bounce)
            pltpu.sync_copy(bounce, sp_acc.at[off])
        plsc.subcore_barrier()

        def gather(j, buf, sem):
            return pltpu.make_async_copy(y2d.at[src_v.at[j]], buf, sem)

        gather(0, buf0, sem0).start()

        def body(i, _):
            j0 = 2 * i
            j1 = 2 * i + 1
            gather(j1, buf1, sem1).start()
            gather(j0, buf0, sem0).wait()
            pltpu.sync_copy(buf0, sp_acc.at[dst_v.at[j0]], add=True)
            gather(j1 + 1, buf0, sem0).start()
            gather(j1, buf1, sem1).wait()
            pltpu.sync_copy(buf1, sp_acc.at[dst_v.at[j1]], add=True)
            return None
        lax.fori_loop(0, (ESTEPS - 1) // 2, body, None)
        jlast = ESTEPS - 1
        gather(jlast, buf0, sem0).wait()
        pltpu.sync_copy(buf0, sp_acc.at[dst_v.at[jlast]], add=True)

        plsc.subcore_barrier()
        csl = pl.ds(q * CW, CW)
        for p in range(8):
            off = pl.ds(s * ROWS_PT + p * PIECE, PIECE)
            pltpu.sync_copy(sp_acc.at[off], bounce)
            pltpu.sync_copy(bounce, out.at[off, csl])
        plsc.subcore_barrier()

    ys = (y0, y1, y2, y3, y4, y5, y6, y7,
          y8, y9, y10, y11, y12, y13, y14, y15)

    @pl.when(c == 0)
    def _():
        for q in range(8):
            run_chunk(ys[q], q)

    @pl.when(c == 1)
    def _():
        for q in range(8, 16):
            run_chunk(ys[q], q)


@functools.partial(
    pl.kernel,
    out_type=jax.ShapeDtypeStruct((DPAD, DP), _f32),
    mesh=_MESH,
    compiler_params=_SC_PARAMS,
    scratch_types=(
        pltpu.VMEM((DSTEPS, 128), jnp.int32),
        pltpu.VMEM((DSTEPS, 128), jnp.int32),
        pltpu.VMEM((128, DP), _f32),
        pltpu.VMEM((128, DP), _f32),
        pltpu.SemaphoreType.DMA,
        pltpu.SemaphoreType.DMA,
    ),
)
def _sc_decode(qa, qb, el0_r, el1_r, x1p, i0_v, i1_v, bufa, bufb, sema, semb):
    c = lax.axis_index("c")
    s = lax.axis_index("s")
    w = c * 16 + s
    base = w * (DSTEPS * 128)

    pltpu.sync_copy(el0_r.at[pl.ds(w * DSTEPS, DSTEPS)], i0_v)
    pltpu.sync_copy(el1_r.at[pl.ds(w * DSTEPS, DSTEPS)], i1_v)

    def step(j, _):
        pltpu.async_copy(qa.at[i0_v.at[j]], bufa, sema)
        pltpu.async_copy(qb.at[i1_v.at[j]], bufb, semb)
        pltpu.make_async_copy(qa.at[i0_v.at[j]], bufa, sema).wait()
        pltpu.make_async_copy(qb.at[i1_v.at[j]], bufb, semb).wait()

        def row(r, _):
            for cc in range(DP // 16):
                sl = pl.ds(cc * 16, 16)
                bufa[r, sl] = jnp.maximum(bufa[r, sl] + bufb[r, sl], 0.0)
            return None
        lax.fori_loop(0, 128, row, None)
        pltpu.sync_copy(bufa, x1p.at[pl.ds(base + j * 128, 128)])
        return None
    lax.fori_loop(0, DSTEPS, step, None)


_RB = 400


def _chunk_outspecs():
    specs = [pl.BlockSpec((_RB, CW), lambda i: (i, 0)) for _ in range(NCH)]
    shapes = [jax.ShapeDtypeStruct((SPROWS, CW), _f32) for _ in range(NCH)]
    return specs, shapes


def _tc_y1_body(x_ref, degt_ref, w_ref, *outs):
    y_refs = outs[:NCH]
    dinv_ref = outs[NCH]
    deg = degt_ref[:, 0] + degt_ref[:, 1] + 1.0
    dinv = lax.rsqrt(deg)
    xw = jnp.dot(x_ref[...], w_ref[...], preferred_element_type=_f32)
    y = xw * dinv[:, None]
    for q in range(NCH):
        y_refs[q][...] = y[:, q * CW:(q + 1) * CW]
    dinv_ref[...] = dinv[:, None]


def _tc_y1(x, degt, w1p):
    cspecs, cshapes = _chunk_outspecs()
    return pl.pallas_call(
        _tc_y1_body,
        grid=(N // _RB,),
        in_specs=[
            pl.BlockSpec((_RB, D), lambda i: (i, 0)),
            pl.BlockSpec((_RB, 2), lambda i: (i, 0)),
            pl.BlockSpec((D, DP), lambda i: (0, 0)),
        ],
        out_specs=cspecs + [pl.BlockSpec((_RB, 1), lambda i: (i, 0))],
        out_shape=cshapes + [jax.ShapeDtypeStruct((N, 1), _f32)],
    )(x, degt, w1p)


def _tc_h_body(acc_ref, dinv_ref, b_ref, ca_ref, cb_ref,
               h_ref, sab_ref, cross_ref):
    hb = jnp.maximum(acc_ref[...] * dinv_ref[...] + b_ref[...], 0.0)
    h_ref[...] = hb
    ca = ca_ref[...]
    cb = cb_ref[...]
    sa = jnp.dot(ca.T, hb, preferred_element_type=_f32)
    sb = jnp.dot(cb.T, hb, preferred_element_type=_f32)
    sab = jnp.concatenate([sa, sb], axis=0)
    fa = (ca > 0.0).astype(_f32)
    crossblk = jnp.sum(fa * cb)
    col = lax.broadcasted_iota(jnp.int32, (1, DP), 1)
    crossmat = jnp.where(col == 0, crossblk, 0.0)

    @pl.when(pl.program_id(0) == 0)
    def _():
        sab_ref[...] = sab
        cross_ref[...] = crossmat

    @pl.when(pl.program_id(0) != 0)
    def _():
        sab_ref[...] += sab
        cross_ref[...] += crossmat


def _tc_h(accT, dinvc, b1p, ca2, cb2):
    return pl.pallas_call(
        _tc_h_body,
        grid=(N // _RB,),
        in_specs=[
            pl.BlockSpec((_RB, DP), lambda i: (i, 0)),
            pl.BlockSpec((_RB, 1), lambda i: (i, 0)),
            pl.BlockSpec((1, DP), lambda i: (0, 0)),
            pl.BlockSpec((_RB, 1), lambda i: (i, 0)),
            pl.BlockSpec((_RB, 1), lambda i: (i, 0)),
        ],
        out_specs=[
            pl.BlockSpec((_RB, DP), lambda i: (i, 0)),
            pl.BlockSpec((2, DP), lambda i: (0, 0)),
            pl.BlockSpec((1, DP), lambda i: (0, 0)),
        ],
        out_shape=[
            jax.ShapeDtypeStruct((N, DP), _f32),
            jax.ShapeDtypeStruct((2, DP), _f32),
            jax.ShapeDtypeStruct((1, DP), _f32),
        ],
    )(accT, dinvc, b1p, ca2, cb2)


def _tc_y2_body(h_ref, ca_ref, cb_ref, vab_ref, w_ref, dinv_ref, *y_refs):
    fa = (ca_ref[...] > 0.0).astype(_f32)
    fb = (cb_ref[...] > 0.0).astype(_f32)
    h2 = h_ref[...] + fa * vab_ref[0:1, :] + fb * vab_ref[1:2, :]
    y2 = jnp.dot(h2, w_ref[...],
                 preferred_element_type=_f32) * dinv_ref[...]
    for q in range(NCH):
        y_refs[q][...] = y2[:, q * CW:(q + 1) * CW]


def _tc_y2(h, ca2, cb2, vab, w2p, dinvc):
    cspecs, cshapes = _chunk_outspecs()
    return pl.pallas_call(
        _tc_y2_body,
        grid=(N // _RB,),
        in_specs=[
            pl.BlockSpec((_RB, DP), lambda i: (i, 0)),
            pl.BlockSpec((_RB, 1), lambda i: (i, 0)),
            pl.BlockSpec((_RB, 1), lambda i: (i, 0)),
            pl.BlockSpec((2, DP), lambda i: (0, 0)),
            pl.BlockSpec((DP, DP), lambda i: (0, 0)),
            pl.BlockSpec((_RB, 1), lambda i: (i, 0)),
        ],
        out_specs=cspecs,
        out_shape=cshapes,
    )(h, ca2, cb2, vab, w2p, dinvc)


def _tc_qaqb_body(acc_ref, dinv_ref, b_ref, x_ref, wa_ref, wb_ref,
                  qa_ref, qb_ref):
    z = acc_ref[...] * dinv_ref[...] + b_ref[...]
    p = (z + x_ref[...]) * 0.5
    qa_ref[...] = jnp.dot(p, wa_ref[...], preferred_element_type=_f32)
    qb_ref[...] = jnp.dot(p, wb_ref[...], preferred_element_type=_f32)


def _tc_qaqb(acc2T, dinvc, b2p, xp, wap, wbp):
    return pl.pallas_call(
        _tc_qaqb_body,
        grid=(N // _RB,),
        in_specs=[
            pl.BlockSpec((_RB, DP), lambda i: (i, 0)),
            pl.BlockSpec((_RB, 1), lambda i: (i, 0)),
            pl.BlockSpec((1, DP), lambda i: (0, 0)),
            pl.BlockSpec((_RB, DP), lambda i: (i, 0)),
            pl.BlockSpec((DP, DP), lambda i: (0, 0)),
            pl.BlockSpec((DP, DP), lambda i: (0, 0)),
        ],
        out_specs=[
            pl.BlockSpec((_RB, DP), lambda i: (i, 0)),
            pl.BlockSpec((_RB, DP), lambda i: (i, 0)),
        ],
        out_shape=[
            jax.ShapeDtypeStruct((N, DP), _f32),
            jax.ShapeDtypeStruct((N, DP), _f32),
        ],
    )(acc2T, dinvc, b2p, xp, wap, wbp)


_RB2 = 1000


def _tc_score_body(x1_ref, w_ref, sc_ref, x1o_ref):
    x1 = x1_ref[...]
    sc_ref[...] = jnp.dot(x1, w_ref[...], preferred_element_type=_f32)
    x1o_ref[...] = x1[:, :D]


def _tc_score(x1p, wl2p):
    return pl.pallas_call(
        _tc_score_body,
        grid=(EL // _RB2,),
        in_specs=[
            pl.BlockSpec((_RB2, DP), lambda i: (i, 0)),
            pl.BlockSpec((DP, 8), lambda i: (0, 0)),
        ],
        out_specs=[
            pl.BlockSpec((_RB2, 8), lambda i: (i, 0)),
            pl.BlockSpec((_RB2, D), lambda i: (i, 0)),
        ],
        out_shape=[
            jax.ShapeDtypeStruct((EL, 8), _f32),
            jax.ShapeDtypeStruct((EL, D), _f32),
        ],
    )(x1p, wl2p)


def _pad_to(a, n, val):
    return jnp.concatenate(
        [a, jnp.full((n - a.shape[0],), val, a.dtype)])


def kernel(x, edge_index, type_lnc_idx, type_mi_idx, edge_label_index,
           W1c, b1c, W2c, b2c, Wl1, Wl2):
    src = edge_index[0]
    dst = edge_index[1]

    dst_c = _pad_to(dst, CPAD32, TRASH).reshape(32 * CSTEPS, 128)
    lnc_r = _pad_to(type_lnc_idx, IPAD, TRASH).reshape(16 * ISTEPS, 128)
    mi_r = _pad_to(type_mi_idx, IPAD, TRASH).reshape(16 * ISTEPS, 128)
    src_r = _pad_to(src, 16 * EPAD, 0).reshape(16 * ESTEPS, 128)
    dst_r = _pad_to(dst, 16 * EPAD, TRASH).reshape(16 * ESTEPS, 128)
    el0_r = _pad_to(edge_label_index[0], DPAD, 0).reshape(32 * DSTEPS, 128)
    el1_r = _pad_to(edge_label_index[1], DPAD, 0).reshape(32 * DSTEPS, 128)

    w1p = jnp.zeros((D, DP), _f32).at[:, :D].set(W1c)
    w2p = jnp.zeros((DP, DP), _f32).at[:D, :D].set(W2c)
    b1p = jnp.zeros((1, DP), _f32).at[0, :D].set(b1c)
    b2p = jnp.zeros((1, DP), _f32).at[0, :D].set(b2c)
    wap = jnp.zeros((DP, DP), _f32).at[:D, :D].set(Wl1[:D])
    wbp = jnp.zeros((DP, DP), _f32).at[:D, :D].set(Wl1[D:])
    wl2p = jnp.zeros((DP, 8), _f32).at[:D, 0:1].set(Wl2)
    xp = jnp.zeros((N, DP), _f32).at[:, :D].set(x)

    degp, cnta, cntb = _sc_counts(dst_c, lnc_r, mi_r)
    degt = jnp.transpose(degp.reshape(2, SPROWS)[:, :N], (1, 0))
    ca2 = cnta[:N, None]
    cb2 = cntb[:N, None]

    *y1c, dinvc = _tc_y1(x, degt, w1p)
    acc1T = _sc_edge(*y1c, src_r, dst_r)[:N]
    h, sab, crossm = _tc_h(acc1T, dinvc, b1p, ca2, cb2)

    va = sab[0] / 10000.0
    cross = crossm[0, 0]
    vb = sab[1] / 10000.0 + va * (cross / 10000.0)
    vab = jnp.stack([va, vb])

    y2c = _tc_y2(h, ca2, cb2, vab, w2p, dinvc)
    acc2T = _sc_edge(*y2c, src_r, dst_r)[:N]

    qa, qb = _tc_qaqb(acc2T, dinvc, b2p, xp, wap, wbp)
    x1pad = _sc_decode(qa, qb, el0_r, el1_r)
    x1full = x1pad[:EL]
    score8, x1 = _tc_score(x1full, wl2p)
    return score8[:, 0], x1
